# baseline (speedup 1.0000x reference)
import functools

import jax
import jax.numpy as jnp
from jax import lax
from jax.experimental import pallas as pl
from jax.experimental.pallas import tpu as pltpu
from jax.experimental.pallas import tpu_sc as plsc

B = 4
SEQ = 2048
D = 768

_info = plsc.get_sparse_core_info()
_NC = _info.num_cores
_NS = _info.num_subcores
_NW = _NC * _NS
_ROWS = SEQ // _NW

_mesh = plsc.VectorSubcoreMesh(core_axis_name="c", subcore_axis_name="s")


_SC_ROWS = _NS * _ROWS
_QROWS = _SC_ROWS // B


@functools.partial(
    pl.kernel,
    mesh=_mesh,
    out_type=jax.ShapeDtypeStruct((B, SEQ, D), jnp.float32),
    scratch_types=[pltpu.VMEM_SHARED((_SC_ROWS, D), jnp.float32)],
)
def _sc_copy(tab_hbm, out_hbm, shared):
    c = lax.axis_index("c")
    s = lax.axis_index("s")
    sc_base = c * _SC_ROWS
    pltpu.sync_copy(
        tab_hbm.at[pl.ds(sc_base + s * _ROWS, _ROWS)],
        shared.at[pl.ds(s * _ROWS, _ROWS)],
    )
    plsc.subcore_barrier()
    q = s % B
    b = s // B
    pltpu.sync_copy(
        shared.at[pl.ds(q * _QROWS, _QROWS)],
        out_hbm.at[b, pl.ds(sc_base + q * _QROWS, _QROWS)],
    )


def kernel(x, pos_table):
    del x
    return _sc_copy(pos_table)
